# Initial kernel scaffold; baseline (speedup 1.0000x reference)
import functools

import jax
import jax.numpy as jnp
from jax import lax
from jax.experimental import pallas as pl
from jax.experimental.pallas import tpu as pltpu
from jax.experimental.pallas import tpu_sc as plsc

N_NODES = 10000
D = 256
E = 160000
NC = 2
NS = 16
NW = NC * NS
EW = E // NW
C = 40
NCHUNK = EW // C
LANES = 16
DSTEP = D // LANES


def _body(h_hbm, src_hbm, dst_hbm, out_hbm, idx_s, idx_d, rows_s, rows_d,
          scores, sem):
    wid = lax.axis_index("s") * NC + lax.axis_index("c")
    base_w = wid * EW

    def chunk_body(ci, carry):
        base = base_w + ci * C
        pltpu.sync_copy(src_hbm.at[pl.ds(base, C)], idx_s)
        pltpu.sync_copy(dst_hbm.at[pl.ds(base, C)], idx_d)
        cp_s = pltpu.async_copy(h_hbm.at[idx_s], rows_s, sem)
        cp_d = pltpu.async_copy(h_hbm.at[idx_d], rows_d, sem)
        cp_s.wait()
        cp_d.wait()

        def edge_body(e, ecarry):
            acc = rows_s[e, pl.ds(0, LANES)] * rows_d[e, pl.ds(0, LANES)]
            for j in range(1, DSTEP):
                acc = acc + (rows_s[e, pl.ds(j * LANES, LANES)]
                             * rows_d[e, pl.ds(j * LANES, LANES)])
            scores[e] = jnp.sum(acc)
            return ecarry

        lax.fori_loop(0, C, edge_body, 0)
        pltpu.sync_copy(scores, out_hbm.at[pl.ds(base, C)])
        return carry

    lax.fori_loop(0, NCHUNK, chunk_body, 0)


@jax.jit
def _score(h, src, dst):
    kern = pl.kernel(
        _body,
        out_type=jax.ShapeDtypeStruct((E,), jnp.float32),
        mesh=plsc.VectorSubcoreMesh(core_axis_name="c", subcore_axis_name="s"),
        scratch_types=[
            pltpu.VMEM((C,), jnp.int32),
            pltpu.VMEM((C,), jnp.int32),
            pltpu.VMEM((C, D), jnp.float32),
            pltpu.VMEM((C, D), jnp.float32),
            pltpu.VMEM((C,), jnp.float32),
            pltpu.SemaphoreType.DMA,
        ],
    )
    return kern(h, src, dst)


def kernel(h, edge_index):
    src = edge_index[0].astype(jnp.int32)
    dst = edge_index[1].astype(jnp.int32)
    return _score(h, src, dst).reshape(E, 1)


# speedup vs baseline: 1.8790x; 1.8790x over previous
import functools

import jax
import jax.numpy as jnp
from jax import lax
from jax.experimental import pallas as pl
from jax.experimental.pallas import tpu as pltpu
from jax.experimental.pallas import tpu_sc as plsc

N_NODES = 10000
D = 256
E = 160000
NC = 2
NS = 16
NW = NC * NS
EW = E // NW
C = 40
NCHUNK = EW // C
LANES = 16
DSTEP = D // LANES


def _body(h_hbm, src_hbm, dst_hbm, out_hbm, idx_s, idx_d, rows_s, rows_d,
          scores, sem):
    wid = lax.axis_index("s") * NC + lax.axis_index("c")
    base_w = wid * EW
    last_lane = lax.iota(jnp.int32, LANES) == (LANES - 1)

    def chunk_body(ci, carry):
        base = base_w + ci * C
        pltpu.sync_copy(src_hbm.at[pl.ds(base, C)], idx_s)
        pltpu.sync_copy(dst_hbm.at[pl.ds(base, C)], idx_d)
        cp_s = pltpu.async_copy(h_hbm.at[idx_s], rows_s, sem)
        cp_d = pltpu.async_copy(h_hbm.at[idx_d], rows_d, sem)
        cp_s.wait()
        cp_d.wait()

        def edge_body(e, ecarry):
            acc = rows_s[e, pl.ds(0, LANES)] * rows_d[e, pl.ds(0, LANES)]
            for j in range(1, DSTEP):
                acc = acc + (rows_s[e, pl.ds(j * LANES, LANES)]
                             * rows_d[e, pl.ds(j * LANES, LANES)])
            tot = plsc.cumsum(acc)
            plsc.store_scatter(scores, [jnp.full((LANES,), e, jnp.int32)],
                               tot, mask=last_lane)
            return ecarry

        lax.fori_loop(0, C, edge_body, 0)
        pltpu.sync_copy(scores, out_hbm.at[pl.ds(base, C)])
        return carry

    lax.fori_loop(0, NCHUNK, chunk_body, 0)


@jax.jit
def _score(h, src, dst):
    kern = pl.kernel(
        _body,
        out_type=jax.ShapeDtypeStruct((E,), jnp.float32),
        mesh=plsc.VectorSubcoreMesh(core_axis_name="c", subcore_axis_name="s"),
        scratch_types=[
            pltpu.VMEM((C,), jnp.int32),
            pltpu.VMEM((C,), jnp.int32),
            pltpu.VMEM((C, D), jnp.float32),
            pltpu.VMEM((C, D), jnp.float32),
            pltpu.VMEM((C,), jnp.float32),
            pltpu.SemaphoreType.DMA,
        ],
        compiler_params=pltpu.CompilerParams(needs_layout_passes=False),
    )
    return kern(h, src, dst)


def kernel(h, edge_index):
    src = edge_index[0].astype(jnp.int32)
    dst = edge_index[1].astype(jnp.int32)
    return _score(h, src, dst).reshape(E, 1)


# speedup vs baseline: 4.1656x; 2.2170x over previous
import jax
import jax.numpy as jnp
from jax import lax
from jax.experimental import pallas as pl
from jax.experimental.pallas import tpu as pltpu
from jax.experimental.pallas import tpu_sc as plsc

N_NODES = 10000
D = 256
E = 160000
NC = 2
NS = 16
NW = NC * NS
EW = E // NW
C = 40
NCHUNK = EW // C
NPAIR = (NCHUNK + 1) // 2
LANES = 16
DSTEP = D // LANES


def _body(h_hbm, src_hbm, dst_hbm, out_hbm, idx_s, idx_d,
          rows_sa, rows_da, rows_sb, rows_db, scores, sem_a, sem_b):
    wid = lax.axis_index("s") * NC + lax.axis_index("c")
    base_w = wid * EW
    last_lane = lax.iota(jnp.int32, LANES) == (LANES - 1)

    pltpu.sync_copy(src_hbm.at[pl.ds(base_w, EW)], idx_s)
    pltpu.sync_copy(dst_hbm.at[pl.ds(base_w, EW)], idx_d)

    def fire(ci, rs, rd, sem):
        sl = pl.ds(ci * C, C)
        pltpu.async_copy(h_hbm.at[idx_s.at[sl]], rs, sem)
        pltpu.async_copy(h_hbm.at[idx_d.at[sl]], rd, sem)

    def drain(rs, rd, sem):
        pltpu.make_async_copy(h_hbm.at[pl.ds(0, C)], rs, sem).wait()
        pltpu.make_async_copy(h_hbm.at[pl.ds(0, C)], rd, sem).wait()

    def compute(ci, rs, rd):
        def edge_body(e, ecarry):
            accs = [rs[e, pl.ds(j * LANES, LANES)]
                    * rd[e, pl.ds(j * LANES, LANES)] for j in range(4)]
            for j in range(4, DSTEP):
                accs[j % 4] = accs[j % 4] + (rs[e, pl.ds(j * LANES, LANES)]
                                             * rd[e, pl.ds(j * LANES, LANES)])
            acc = (accs[0] + accs[1]) + (accs[2] + accs[3])
            tot = plsc.cumsum(acc)
            plsc.store_scatter(scores,
                               [jnp.full((LANES,), ci * C + e, jnp.int32)],
                               tot, mask=last_lane)
            return ecarry

        lax.fori_loop(0, C, edge_body, 0, unroll=4)

    fire(0, rows_sa, rows_da, sem_a)

    def pair_body(g, carry):
        c0 = 2 * g
        c1 = 2 * g + 1

        @pl.when(c1 < NCHUNK)
        def _():
            fire(c1, rows_sb, rows_db, sem_b)

        drain(rows_sa, rows_da, sem_a)
        compute(c0, rows_sa, rows_da)

        @pl.when(c0 + 2 < NCHUNK)
        def _():
            fire(c0 + 2, rows_sa, rows_da, sem_a)

        @pl.when(c1 < NCHUNK)
        def _():
            drain(rows_sb, rows_db, sem_b)
            compute(c1, rows_sb, rows_db)

        return carry

    lax.fori_loop(0, NPAIR, pair_body, 0)
    pltpu.sync_copy(scores, out_hbm.at[pl.ds(base_w, EW)])


@jax.jit
def _score(h, src, dst):
    kern = pl.kernel(
        _body,
        out_type=jax.ShapeDtypeStruct((E,), jnp.float32),
        mesh=plsc.VectorSubcoreMesh(core_axis_name="c", subcore_axis_name="s"),
        scratch_types=[
            pltpu.VMEM((EW,), jnp.int32),
            pltpu.VMEM((EW,), jnp.int32),
            pltpu.VMEM((C, D), jnp.float32),
            pltpu.VMEM((C, D), jnp.float32),
            pltpu.VMEM((C, D), jnp.float32),
            pltpu.VMEM((C, D), jnp.float32),
            pltpu.VMEM((EW,), jnp.float32),
            pltpu.SemaphoreType.DMA,
            pltpu.SemaphoreType.DMA,
        ],
        compiler_params=pltpu.CompilerParams(needs_layout_passes=False),
    )
    return kern(h, src, dst)


def kernel(h, edge_index):
    src = edge_index[0].astype(jnp.int32)
    dst = edge_index[1].astype(jnp.int32)
    return _score(h, src, dst).reshape(E, 1)
